# baseline (speedup 1.0000x reference)
import jax
import jax.numpy as jnp
from jax import lax
from jax.experimental import pallas as pl
from jax.experimental.pallas import tpu as pltpu, tpu_sc as plsc

_B = 16384
_NW = 32
_BPW = _B // _NW
_CHUNK = 128
_NCHUNK = _BPW // _CHUNK


def _sc_body(x_hbm, gt_hbm, out_hbm, idx_v, idx2_v, dig_v, out_v, sem):
    nc = 2
    wid = lax.axis_index("s") * nc + lax.axis_index("c")
    base = wid * _BPW

    pltpu.sync_copy(x_hbm.at[pl.ds(base, _BPW)], idx_v)

    for i in range(_BPW // 16):
        k, r = divmod(i * 16, _CHUNK)
        v2 = idx_v[pl.ds(i * 16, 16)] * 2
        idx2_v.at[k][pl.ds(r, 16)] = v2
        idx2_v.at[_NCHUNK + k][pl.ds(r, 16)] = v2 + 1

    copies = [
        pltpu.async_copy(gt_hbm.at[idx2_v.at[k]],
                         dig_v.at[pl.ds(k * _CHUNK, _CHUNK)], sem)
        for k in range(2 * _NCHUNK)
    ]
    for c in copies:
        c.wait()

    lanes = lax.iota(jnp.int32, 16)
    shift = (lanes & ~3) + 3 - (lanes % 4)

    def body(i, carry):
        d1 = dig_v[pl.ds(i * 16, 16)]
        d2 = dig_v[pl.ds(_BPW + i * 16, 16)]
        for j in range(8):
            packed = (d1[2 * j] | (d2[2 * j] << 4)
                      | (d1[2 * j + 1] << 8) | (d2[2 * j + 1] << 12))
            bits = ((packed >> shift) & 1).astype(jnp.float32)
            out_v[pl.ds(i * 128 + j * 16, 16)] = bits
        return carry

    lax.fori_loop(0, _BPW // 16, body, 0, unroll=4)

    pltpu.sync_copy(out_v, out_hbm.at[pl.ds(base * 8, _BPW * 8)])


def kernel(x, ground_truth):
    gt_flat = ground_truth.reshape(-1)
    mesh = plsc.VectorSubcoreMesh(core_axis_name="c", subcore_axis_name="s",
                                  num_cores=2, num_subcores=16)
    out_flat = pl.kernel(
        _sc_body,
        out_type=jax.ShapeDtypeStruct((_B * 8,), jnp.float32),
        mesh=mesh,
        scratch_types=[
            pltpu.VMEM((_BPW,), jnp.int32),
            pltpu.VMEM((2 * _NCHUNK, _CHUNK), jnp.int32),
            pltpu.VMEM((2 * _BPW,), jnp.int32),
            pltpu.VMEM((_BPW * 8,), jnp.float32),
            pltpu.SemaphoreType.DMA,
        ],
    )(x, gt_flat)
    return out_flat.reshape(_B, 8)


# speedup vs baseline: 16.4945x; 16.4945x over previous
import jax
import jax.numpy as jnp
from jax import lax
from jax.experimental import pallas as pl
from jax.experimental.pallas import tpu as pltpu, tpu_sc as plsc

_B = 16384
_NW = 32
_BPW = _B // _NW
_CHUNK = 128
_NCHUNK = _BPW // _CHUNK


def _sc_body(x_hbm, d1_hbm, d2_hbm, out_hbm, idx_v, dig_v, out_v, sem):
    nc = 2
    wid = lax.axis_index("s") * nc + lax.axis_index("c")
    base = wid * _BPW

    for k in range(_NCHUNK):
        pltpu.sync_copy(x_hbm.at[pl.ds(base + k * _CHUNK, _CHUNK)], idx_v.at[k])
    copies = []
    for k in range(_NCHUNK):
        copies.append(pltpu.async_copy(
            d1_hbm.at[idx_v.at[k]],
            dig_v.at[pl.ds(k * _CHUNK, _CHUNK)], sem))
        copies.append(pltpu.async_copy(
            d2_hbm.at[idx_v.at[k]],
            dig_v.at[pl.ds(_BPW + k * _CHUNK, _CHUNK)], sem))
    for c in copies:
        c.wait()

    lanes = lax.iota(jnp.int32, 16)
    shift = (lanes & ~3) + 3 - (lanes % 4)

    def body(i, carry):
        d1 = dig_v[pl.ds(i * 16, 16)]
        d2 = dig_v[pl.ds(_BPW + i * 16, 16)]
        for j in range(8):
            packed = (d1[2 * j] | (d2[2 * j] << 4)
                      | (d1[2 * j + 1] << 8) | (d2[2 * j + 1] << 12))
            bits = ((packed >> shift) & 1).astype(jnp.float32)
            out_v[pl.ds(i * 128 + j * 16, 16)] = bits
        return carry

    lax.fori_loop(0, _BPW // 16, body, 0, unroll=4)

    pltpu.sync_copy(out_v, out_hbm.at[pl.ds(base * 8, _BPW * 8)])


def kernel(x, ground_truth):
    d1 = ground_truth[:, 0]
    d2 = ground_truth[:, 1]
    mesh = plsc.VectorSubcoreMesh(core_axis_name="c", subcore_axis_name="s",
                                  num_cores=2, num_subcores=16)
    out_flat = pl.kernel(
        _sc_body,
        out_type=jax.ShapeDtypeStruct((_B * 8,), jnp.float32),
        mesh=mesh,
        scratch_types=[
            pltpu.VMEM((_NCHUNK, _CHUNK), jnp.int32),
            pltpu.VMEM((2 * _BPW,), jnp.int32),
            pltpu.VMEM((_BPW * 8,), jnp.float32),
            pltpu.SemaphoreType.DMA,
        ],
    )(x, d1, d2)
    return out_flat.reshape(_B, 8)


# speedup vs baseline: 17.0741x; 1.0351x over previous
import jax
import jax.numpy as jnp
from jax import lax
from jax.experimental import pallas as pl
from jax.experimental.pallas import tpu as pltpu, tpu_sc as plsc

_B = 16384
_NW = 32
_BPW = _B // _NW
_CHUNK = 128
_NCHUNK = _BPW // _CHUNK


def _sc_body(x_hbm, p_hbm, out_hbm, idx_v, dig_v, out_v, sem):
    nc = 2
    wid = lax.axis_index("s") * nc + lax.axis_index("c")
    base = wid * _BPW

    for k in range(_NCHUNK):
        pltpu.sync_copy(x_hbm.at[pl.ds(base + k * _CHUNK, _CHUNK)], idx_v.at[k])
    copies = [
        pltpu.async_copy(p_hbm.at[idx_v.at[k]],
                         dig_v.at[pl.ds(k * _CHUNK, _CHUNK)], sem)
        for k in range(_NCHUNK)
    ]
    for c in copies:
        c.wait()

    lanes = lax.iota(jnp.int32, 16)
    shift = (lanes & ~3) + 3 - (lanes % 4)

    def body(i, carry):
        v = dig_v[pl.ds(i * 16, 16)]
        for j in range(8):
            pw = v[2 * j] | (v[2 * j + 1] << 8)
            bits = ((pw >> shift) & 1).astype(jnp.float32)
            out_v[pl.ds(i * 128 + j * 16, 16)] = bits
        return carry

    lax.fori_loop(0, _BPW // 16, body, 0, unroll=4)

    pltpu.sync_copy(out_v, out_hbm.at[pl.ds(base * 8, _BPW * 8)])


def kernel(x, ground_truth):
    packed = ground_truth[:, 0] | (ground_truth[:, 1] << 4)
    mesh = plsc.VectorSubcoreMesh(core_axis_name="c", subcore_axis_name="s",
                                  num_cores=2, num_subcores=16)
    out_flat = pl.kernel(
        _sc_body,
        out_type=jax.ShapeDtypeStruct((_B * 8,), jnp.float32),
        mesh=mesh,
        scratch_types=[
            pltpu.VMEM((_NCHUNK, _CHUNK), jnp.int32),
            pltpu.VMEM((_BPW,), jnp.int32),
            pltpu.VMEM((_BPW * 8,), jnp.float32),
            pltpu.SemaphoreType.DMA,
        ],
    )(x, packed)
    return out_flat.reshape(_B, 8)
